# baseline (speedup 1.0000x reference)
import functools

import jax
import jax.numpy as jnp
from jax import lax
from jax.experimental import pallas as pl
from jax.experimental.pallas import tpu as pltpu
from jax.experimental.pallas import tpu_sc as plsc

VOCAB = 1000000
EMB = 64
IMG_FEAT = 2048
HIDDEN = 256
N_WORDS = 819200

_NC = 2
_NS = 16
_NW = _NC * _NS
_PER_W = N_WORDS // _NW
_CHUNK = 128
_NCHUNK = _PER_W // _CHUNK


def _sc_bagsum(idx_hbm, table_hbm, out_hbm, idx_v, rows_v, acc_v, sem):
    c = lax.axis_index("c")
    s = lax.axis_index("s")
    wid = s * _NC + c

    pltpu.sync_copy(idx_hbm.at[wid], idx_v)

    zeros = jnp.zeros((16,), jnp.float32)

    def chunk_body(i, carry):
        pltpu.async_copy(table_hbm.at[idx_v.at[i]], rows_v, sem).wait()

        def row_body(r, cc):
            a0, a1, a2, a3 = cc
            a0 = a0 + rows_v[r, pl.ds(0, 16)]
            a1 = a1 + rows_v[r, pl.ds(16, 16)]
            a2 = a2 + rows_v[r, pl.ds(32, 16)]
            a3 = a3 + rows_v[r, pl.ds(48, 16)]
            return (a0, a1, a2, a3)

        return lax.fori_loop(0, _CHUNK, row_body, carry)

    a0, a1, a2, a3 = lax.fori_loop(0, _NCHUNK, chunk_body, (zeros,) * 4)

    acc_v[pl.ds(0, 16)] = a0
    acc_v[pl.ds(16, 16)] = a1
    acc_v[pl.ds(32, 16)] = a2
    acc_v[pl.ds(48, 16)] = a3
    pltpu.sync_copy(acc_v, out_hbm.at[wid])


_sc_bagsum_call = pl.kernel(
    _sc_bagsum,
    out_type=jax.ShapeDtypeStruct((_NW, EMB), jnp.float32),
    scratch_types=[
        pltpu.VMEM((_NCHUNK, _CHUNK), jnp.int32),
        pltpu.VMEM((_CHUNK, EMB), jnp.float32),
        pltpu.VMEM((EMB,), jnp.float32),
        pltpu.SemaphoreType.DMA,
    ],
    mesh=plsc.VectorSubcoreMesh(core_axis_name="c", subcore_axis_name="s"),
    compiler_params=pltpu.CompilerParams(use_tc_tiling_on_sc=False),
)


def _mlp(part_ref, img_ref, wh_ref, bh_ref, wo_ref, bo_ref, out_ref):
    bow = jnp.sum(part_ref[...], axis=0, keepdims=True)
    h = (
        jnp.dot(bow, wh_ref[:EMB, :], preferred_element_type=jnp.float32)
        + jnp.dot(img_ref[...], wh_ref[EMB:, :],
                  preferred_element_type=jnp.float32)
        + bh_ref[...]
    )
    o = jnp.dot(h, wo_ref[...], preferred_element_type=jnp.float32) + bo_ref[...]
    out_ref[...] = jax.nn.sigmoid(o)


_mlp_call = pl.pallas_call(
    _mlp,
    out_shape=jax.ShapeDtypeStruct((1, 1), jnp.float32),
)


def kernel(word_inputs, image_inputs, emb_table, W_h, b_h, W_o, b_o):
    idx3 = word_inputs.astype(jnp.int32).reshape(_NW, _NCHUNK, _CHUNK)
    partials = _sc_bagsum_call(idx3, emb_table)
    prob = _mlp_call(
        partials,
        image_inputs.reshape(1, IMG_FEAT),
        W_h,
        b_h.reshape(1, HIDDEN),
        W_o,
        b_o.reshape(1, 1),
    )
    return prob.reshape((1,))


# speedup vs baseline: 1.2867x; 1.2867x over previous
import functools

import jax
import jax.numpy as jnp
from jax import lax
from jax.experimental import pallas as pl
from jax.experimental.pallas import tpu as pltpu
from jax.experimental.pallas import tpu_sc as plsc

VOCAB = 1000000
EMB = 64
IMG_FEAT = 2048
HIDDEN = 256
N_WORDS = 819200

_NC = 2
_NS = 16
_NW = _NC * _NS
_PER_W = N_WORDS // _NW
_CHUNK = 128
_NCHUNK = _PER_W // _CHUNK


_NBUF = 4


def _sc_bagsum(idx_hbm, table_hbm, out_hbm, idx_v, r0, r1, r2, r3, acc_v,
               s0, s1, s2, s3):
    c = lax.axis_index("c")
    s = lax.axis_index("s")
    wid = s * _NC + c
    bufs = (r0, r1, r2, r3)
    sems = (s0, s1, s2, s3)

    pltpu.sync_copy(idx_hbm.at[wid], idx_v)

    for b in range(_NBUF):
        pltpu.async_copy(table_hbm.at[idx_v.at[b]], bufs[b], sems[b])

    zeros = jnp.zeros((16,), jnp.float32)

    def outer_body(j, carry):
        for b in range(_NBUF):
            i = j * _NBUF + b
            rows_v = bufs[b]
            pltpu.make_async_copy(
                table_hbm.at[pl.ds(0, _CHUNK)], rows_v, sems[b]).wait()

            def row_body(r, cc, rows_v=rows_v):
                a0, a1, a2, a3 = cc
                a0 = a0 + rows_v[r, pl.ds(0, 16)]
                a1 = a1 + rows_v[r, pl.ds(16, 16)]
                a2 = a2 + rows_v[r, pl.ds(32, 16)]
                a3 = a3 + rows_v[r, pl.ds(48, 16)]
                return (a0, a1, a2, a3)

            carry = lax.fori_loop(0, _CHUNK, row_body, carry, unroll=8)

            @pl.when(j < _NCHUNK // _NBUF - 1)
            def _(b=b, i=i, rows_v=rows_v):
                pltpu.async_copy(
                    table_hbm.at[idx_v.at[i + _NBUF]], rows_v, sems[b])

        return carry

    a0, a1, a2, a3 = lax.fori_loop(
        0, _NCHUNK // _NBUF, outer_body, (zeros,) * 4)

    acc_v[pl.ds(0, 16)] = a0
    acc_v[pl.ds(16, 16)] = a1
    acc_v[pl.ds(32, 16)] = a2
    acc_v[pl.ds(48, 16)] = a3
    pltpu.sync_copy(acc_v, out_hbm.at[wid])


_sc_bagsum_call = pl.kernel(
    _sc_bagsum,
    out_type=jax.ShapeDtypeStruct((_NW, EMB), jnp.float32),
    scratch_types=[
        pltpu.VMEM((_NCHUNK, _CHUNK), jnp.int32),
        pltpu.VMEM((_CHUNK, EMB), jnp.float32),
        pltpu.VMEM((_CHUNK, EMB), jnp.float32),
        pltpu.VMEM((_CHUNK, EMB), jnp.float32),
        pltpu.VMEM((_CHUNK, EMB), jnp.float32),
        pltpu.VMEM((EMB,), jnp.float32),
        pltpu.SemaphoreType.DMA,
        pltpu.SemaphoreType.DMA,
        pltpu.SemaphoreType.DMA,
        pltpu.SemaphoreType.DMA,
    ],
    mesh=plsc.VectorSubcoreMesh(core_axis_name="c", subcore_axis_name="s"),
    compiler_params=pltpu.CompilerParams(use_tc_tiling_on_sc=False),
)


def _mlp(part_ref, img_ref, wh_ref, bh_ref, wo_ref, bo_ref, out_ref):
    bow = jnp.sum(part_ref[...], axis=0, keepdims=True)
    h = (
        jnp.dot(bow, wh_ref[:EMB, :], preferred_element_type=jnp.float32)
        + jnp.dot(img_ref[...], wh_ref[EMB:, :],
                  preferred_element_type=jnp.float32)
        + bh_ref[...]
    )
    o = jnp.dot(h, wo_ref[...], preferred_element_type=jnp.float32) + bo_ref[...]
    out_ref[...] = jax.nn.sigmoid(o)


_mlp_call = pl.pallas_call(
    _mlp,
    out_shape=jax.ShapeDtypeStruct((1, 1), jnp.float32),
)


def kernel(word_inputs, image_inputs, emb_table, W_h, b_h, W_o, b_o):
    idx3 = word_inputs.astype(jnp.int32).reshape(_NW, _NCHUNK, _CHUNK)
    partials = _sc_bagsum_call(idx3, emb_table)
    prob = _mlp_call(
        partials,
        image_inputs.reshape(1, IMG_FEAT),
        W_h,
        b_h.reshape(1, HIDDEN),
        W_o,
        b_o.reshape(1, 1),
    )
    return prob.reshape((1,))


# speedup vs baseline: 4.9704x; 3.8628x over previous
import jax
import jax.numpy as jnp
from jax import lax
from jax.experimental import pallas as pl
from jax.experimental.pallas import tpu as pltpu
from jax.experimental.pallas import tpu_sc as plsc

VOCAB = 1000000
EMB = 64
IMG_FEAT = 2048
HIDDEN = 256
N_WORDS = 819200

_NC = 2
_NS = 16
_NW = _NC * _NS
_PER_W = N_WORDS // _NW
_CHUNK = 128
_NCHUNK = _PER_W // _CHUNK
_GROUP = 25
_NGROUP = _NCHUNK // _GROUP

_ZLEN = 25600
_ZCOPY = 25000
_STRIPE = VOCAB // 8


def _sc_hist(idx_hbm, counts_hbm, idx_v, zbuf, ones_v, csh, sem):
    c = lax.axis_index("c")
    s = lax.axis_index("s")
    wid = c * _NS + s

    pltpu.sync_copy(idx_hbm.at[wid], idx_v)

    z16 = jnp.zeros((16,), jnp.float32)
    o16 = jnp.ones((16,), jnp.float32)

    def zfill(i, _):
        zbuf[pl.ds(i * 16, 16)] = z16
        return 0

    lax.fori_loop(0, _ZLEN // 16, zfill, 0, unroll=8)
    for k in range(_CHUNK // 16):
        ones_v[pl.ds(k * 16, 16)] = o16

    @pl.when(s < 8)
    def _():
        for k in range(_STRIPE // _ZCOPY):
            pltpu.sync_copy(
                zbuf.at[pl.ds(0, _ZCOPY)],
                csh.at[pl.ds(s * _STRIPE + k * _ZCOPY, _ZCOPY)],
            )

    plsc.subcore_barrier()

    def group_body(j, _):
        for k in range(_GROUP):
            pltpu.async_copy(
                ones_v, csh.at[idx_v.at[j * _GROUP + k]], sem, add=True)
        pltpu.make_async_copy(
            counts_hbm.at[c].at[pl.ds(0, _GROUP * _CHUNK)],
            zbuf.at[pl.ds(0, _GROUP * _CHUNK)],
            sem,
        ).wait()
        return 0

    lax.fori_loop(0, _NGROUP, group_body, 0)

    plsc.subcore_barrier()

    @pl.when(s < 8)
    def _():
        pltpu.sync_copy(
            csh.at[pl.ds(s * _STRIPE, _STRIPE)],
            counts_hbm.at[c].at[pl.ds(s * _STRIPE, _STRIPE)],
        )


_sc_hist_call = pl.kernel(
    _sc_hist,
    out_type=jax.ShapeDtypeStruct((_NC, VOCAB), jnp.float32),
    scratch_types=[
        pltpu.VMEM((_NCHUNK, _CHUNK), jnp.int32),
        pltpu.VMEM((_ZLEN,), jnp.float32),
        pltpu.VMEM((_CHUNK,), jnp.float32),
        pltpu.VMEM_SHARED((VOCAB,), jnp.float32),
        pltpu.SemaphoreType.DMA,
    ],
    mesh=plsc.VectorSubcoreMesh(core_axis_name="c", subcore_axis_name="s"),
    compiler_params=pltpu.CompilerParams(use_tc_tiling_on_sc=False),
)


_BK = 8192
_NBLK = (VOCAB + _BK - 1) // _BK


def _weighted_sum(tab_ref, cnt_ref, out_ref, acc_ref):
    k = pl.program_id(0)

    @pl.when(k == 0)
    def _():
        acc_ref[...] = jnp.zeros_like(acc_ref)

    cnt = cnt_ref[0:1, :] + cnt_ref[1:2, :]
    i = lax.broadcasted_iota(jnp.int32, (1, _BK), 1)
    cnt = jnp.where(k * _BK + i < VOCAB, cnt, 0.0)
    acc_ref[...] += tab_ref[...] * cnt

    @pl.when(k == _NBLK - 1)
    def _():
        out_ref[...] = jnp.sum(acc_ref[...], axis=1, keepdims=True)


_weighted_sum_call = pl.pallas_call(
    _weighted_sum,
    grid=(_NBLK,),
    in_specs=[
        pl.BlockSpec((EMB, _BK), lambda k: (0, k)),
        pl.BlockSpec((_NC, _BK), lambda k: (0, k)),
    ],
    out_specs=pl.BlockSpec((EMB, 1), lambda k: (0, 0)),
    out_shape=jax.ShapeDtypeStruct((EMB, 1), jnp.float32),
    scratch_shapes=[pltpu.VMEM((EMB, _BK), jnp.float32)],
)


def _mlp(bow_ref, img_ref, wh_ref, bh_ref, wo_ref, bo_ref, out_ref):
    h = (
        jnp.dot(bow_ref[...], wh_ref[:EMB, :],
                preferred_element_type=jnp.float32)
        + jnp.dot(img_ref[...], wh_ref[EMB:, :],
                  preferred_element_type=jnp.float32)
        + bh_ref[...]
    )
    o = jnp.dot(h, wo_ref[...], preferred_element_type=jnp.float32) + bo_ref[...]
    out_ref[...] = jax.nn.sigmoid(o)


_mlp_call = pl.pallas_call(
    _mlp,
    out_shape=jax.ShapeDtypeStruct((1, 1), jnp.float32),
)


def kernel(word_inputs, image_inputs, emb_table, W_h, b_h, W_o, b_o):
    idx3 = word_inputs.astype(jnp.int32).reshape(_NW, _NCHUNK, _CHUNK)
    counts = _sc_hist_call(idx3)
    bow_col = _weighted_sum_call(emb_table.T, counts)
    prob = _mlp_call(
        bow_col.reshape(1, EMB),
        image_inputs.reshape(1, IMG_FEAT),
        W_h,
        b_h.reshape(1, HIDDEN),
        W_o,
        b_o.reshape(1, 1),
    )
    return prob.reshape((1,))


# speedup vs baseline: 6.5636x; 1.3205x over previous
import jax
import jax.numpy as jnp
from jax import lax
from jax.experimental import pallas as pl
from jax.experimental.pallas import tpu as pltpu
from jax.experimental.pallas import tpu_sc as plsc

VOCAB = 1000000
EMB = 64
IMG_FEAT = 2048
HIDDEN = 256
N_WORDS = 819200

_NC = 2
_NS = 16
_NW = _NC * _NS
_PER_W = N_WORDS // _NW
_CHUNK = 128
_NCHUNK = _PER_W // _CHUNK
_GROUP = 25
_NGROUP = _NCHUNK // _GROUP

_ZLEN = 25600
_ZCOPY = 25000
_STRIPE = VOCAB // 8


def _sc_hist(idx_hbm, counts_hbm, idx_v, zbuf, ones_v, csh, sem):
    c = lax.axis_index("c")
    s = lax.axis_index("s")
    wid = c * _NS + s

    pltpu.sync_copy(idx_hbm.at[wid], idx_v)

    z16 = jnp.zeros((16,), jnp.float32)
    o16 = jnp.ones((16,), jnp.float32)

    def zfill(i, _):
        zbuf[pl.ds(i * 16, 16)] = z16
        return 0

    lax.fori_loop(0, _ZLEN // 16, zfill, 0, unroll=8)
    for k in range(_CHUNK // 16):
        ones_v[pl.ds(k * 16, 16)] = o16

    @pl.when(s < 8)
    def _():
        for k in range(_STRIPE // _ZCOPY):
            pltpu.sync_copy(
                zbuf.at[pl.ds(0, _ZCOPY)],
                csh.at[pl.ds(s * _STRIPE + k * _ZCOPY, _ZCOPY)],
            )

    plsc.subcore_barrier()

    def group_body(j, _):
        for k in range(_GROUP):
            pltpu.async_copy(
                ones_v, csh.at[idx_v.at[j * _GROUP + k]], sem, add=True)
        pltpu.make_async_copy(
            counts_hbm.at[c].at[pl.ds(0, _GROUP * _CHUNK)],
            zbuf.at[pl.ds(0, _GROUP * _CHUNK)],
            sem,
        ).wait()
        return 0

    lax.fori_loop(0, _NGROUP, group_body, 0)

    plsc.subcore_barrier()

    @pl.when(s < 8)
    def _():
        pltpu.sync_copy(
            csh.at[pl.ds(s * _STRIPE, _STRIPE)],
            counts_hbm.at[c].at[pl.ds(s * _STRIPE, _STRIPE)],
        )


_sc_hist_call = pl.kernel(
    _sc_hist,
    out_type=jax.ShapeDtypeStruct((_NC, VOCAB), jnp.float32),
    scratch_types=[
        pltpu.VMEM((_NCHUNK, _CHUNK), jnp.int32),
        pltpu.VMEM((_ZLEN,), jnp.float32),
        pltpu.VMEM((_CHUNK,), jnp.float32),
        pltpu.VMEM_SHARED((VOCAB,), jnp.float32),
        pltpu.SemaphoreType.DMA,
    ],
    mesh=plsc.VectorSubcoreMesh(core_axis_name="c", subcore_axis_name="s"),
    compiler_params=pltpu.CompilerParams(use_tc_tiling_on_sc=False),
)


_BK = 32768
_NBLK = (VOCAB + _BK - 1) // _BK


def _weighted_sum(tab_ref, cnt_ref, out_ref, acc_ref):
    k = pl.program_id(0)

    @pl.when(k == 0)
    def _():
        acc_ref[...] = jnp.zeros_like(acc_ref)

    cnt = cnt_ref[0:1, :] + cnt_ref[1:2, :]
    i = lax.broadcasted_iota(jnp.int32, (1, _BK), 1)
    cnt = jnp.where(k * _BK + i < VOCAB, cnt, 0.0)
    acc_ref[...] += tab_ref[...] * cnt

    @pl.when(k == _NBLK - 1)
    def _():
        out_ref[...] = jnp.sum(acc_ref[...], axis=1, keepdims=True)


_weighted_sum_call = pl.pallas_call(
    _weighted_sum,
    grid=(_NBLK,),
    in_specs=[
        pl.BlockSpec((EMB, _BK), lambda k: (0, k)),
        pl.BlockSpec((_NC, _BK), lambda k: (0, k)),
    ],
    out_specs=pl.BlockSpec((EMB, 1), lambda k: (0, 0)),
    out_shape=jax.ShapeDtypeStruct((EMB, 1), jnp.float32),
    scratch_shapes=[pltpu.VMEM((EMB, _BK), jnp.float32)],
)


def _mlp(bow_ref, img_ref, wh_ref, bh_ref, wo_ref, bo_ref, out_ref):
    h = (
        jnp.dot(bow_ref[...], wh_ref[:EMB, :],
                preferred_element_type=jnp.float32)
        + jnp.dot(img_ref[...], wh_ref[EMB:, :],
                  preferred_element_type=jnp.float32)
        + bh_ref[...]
    )
    o = jnp.dot(h, wo_ref[...], preferred_element_type=jnp.float32) + bo_ref[...]
    out_ref[...] = jax.nn.sigmoid(o)


_mlp_call = pl.pallas_call(
    _mlp,
    out_shape=jax.ShapeDtypeStruct((1, 1), jnp.float32),
)


def kernel(word_inputs, image_inputs, emb_table, W_h, b_h, W_o, b_o):
    idx3 = word_inputs.astype(jnp.int32).reshape(_NW, _NCHUNK, _CHUNK)
    counts = _sc_hist_call(idx3)
    bow_col = _weighted_sum_call(emb_table.T, counts)
    prob = _mlp_call(
        bow_col.reshape(1, EMB),
        image_inputs.reshape(1, IMG_FEAT),
        W_h,
        b_h.reshape(1, HIDDEN),
        W_o,
        b_o.reshape(1, 1),
    )
    return prob.reshape((1,))
